# baseline (speedup 1.0000x reference)
import functools

import jax
import jax.numpy as jnp
from jax import lax
from jax.experimental import pallas as pl
from jax.experimental.pallas import tpu as pltpu
from jax.experimental.pallas import tpu_sc as plsc

B = 16384
N_CELLS = 64
N_NODES = 66
L = 16

_info = plsc.get_sparse_core_info()
NC, NS = 1, _info.num_subcores
NW = NC * NS
BPW = B // NW


def _sc_interp(cid_hbm, sf_hbm, nv0_hbm, nv1_hbm, conn_hbm, out_hbm,
               cid_v, sf_v, nv0_v, nv1_v, conn_v, out0_v, out1_v, sem):
    wid = lax.axis_index("s") * NC + lax.axis_index("c")
    base = wid * BPW

    cps = [
        pltpu.async_copy(cid_hbm.at[pl.ds(base, BPW)], cid_v, sem),
        pltpu.async_copy(sf_hbm.at[pl.ds(base * 3, BPW * 3)], sf_v, sem),
        pltpu.async_copy(nv0_hbm, nv0_v, sem),
        pltpu.async_copy(nv1_hbm, nv1_v, sem),
        pltpu.async_copy(conn_hbm, conn_v, sem),
    ]
    for cp in cps:
        cp.wait()

    iota3 = lax.iota(jnp.int32, L) * 3

    def body(i, _):
        cid = cid_v[pl.ds(i * L, L)]
        c3 = cid * 3
        r3 = i * (3 * L) + iota3
        acc0 = jnp.zeros((L,), jnp.float32)
        acc1 = jnp.zeros((L,), jnp.float32)
        for k in range(3):
            n = plsc.load_gather(conn_v, [c3 + k]) - 1
            w = plsc.load_gather(sf_v, [r3 + k])
            acc0 += w * plsc.load_gather(nv0_v, [n])
            acc1 += w * plsc.load_gather(nv1_v, [n])
        out0_v[pl.ds(i * L, L)] = acc0
        out1_v[pl.ds(i * L, L)] = acc1
        return _

    Hq = BPW // 2
    lax.fori_loop(0, Hq // L, body, None, unroll=1)
    o0a = pltpu.async_copy(out0_v.at[pl.ds(0, Hq)], out_hbm.at[0, pl.ds(base, Hq)], sem)
    o1a = pltpu.async_copy(out1_v.at[pl.ds(0, Hq)], out_hbm.at[1, pl.ds(base, Hq)], sem)
    lax.fori_loop(Hq // L, BPW // L, body, None, unroll=1)
    o0b = pltpu.async_copy(out0_v.at[pl.ds(Hq, Hq)], out_hbm.at[0, pl.ds(base + Hq, Hq)], sem)
    o1b = pltpu.async_copy(out1_v.at[pl.ds(Hq, Hq)], out_hbm.at[1, pl.ds(base + Hq, Hq)], sem)
    o0a.wait()
    o1a.wait()
    o0b.wait()
    o1b.wait()


@functools.partial(
    pl.kernel,
    mesh=plsc.VectorSubcoreMesh(core_axis_name="c", subcore_axis_name="s", num_cores=1),
    out_type=jax.ShapeDtypeStruct((2, B), jnp.float32),
    compiler_params=pltpu.CompilerParams(needs_layout_passes=False),
    scratch_types=[
        pltpu.VMEM((BPW,), jnp.int32),
        pltpu.VMEM((BPW * 3,), jnp.float32),
        pltpu.VMEM((N_NODES,), jnp.float32),
        pltpu.VMEM((N_NODES,), jnp.float32),
        pltpu.VMEM((N_CELLS * 3,), jnp.int32),
        pltpu.VMEM((BPW,), jnp.float32),
        pltpu.VMEM((BPW,), jnp.float32),
        pltpu.SemaphoreType.DMA,
    ],
)
def _interp_kernel(*refs):
    _sc_interp(*refs)


def kernel(x, cell_id, nodal_values, shape_functions, connectivity):
    del x
    sf_flat = shape_functions.reshape(-1)
    nv0 = nodal_values[0, :, 0]
    nv1 = nodal_values[1, :, 0]
    conn_flat = connectivity.reshape(-1)
    return _interp_kernel(cell_id, sf_flat, nv0, nv1, conn_flat)


# speedup vs baseline: 1.0016x; 1.0016x over previous
import functools

import jax
import jax.numpy as jnp
from jax import lax
from jax.experimental import pallas as pl
from jax.experimental.pallas import tpu as pltpu
from jax.experimental.pallas import tpu_sc as plsc

B = 16384
N_CELLS = 64
N_NODES = 66
L = 16

_info = plsc.get_sparse_core_info()
NC, NS = 1, _info.num_subcores
NW = NC * NS
BPW = B // NW


def _sc_interp(cid_hbm, sf_hbm, nv0_hbm, nv1_hbm, conn_hbm, out_hbm,
               cid_v, sf_v, nv0_v, nv1_v, conn_v, out0_v, out1_v, sem):
    wid = lax.axis_index("s") * NC + lax.axis_index("c")
    base = wid * BPW

    cps = [
        pltpu.async_copy(cid_hbm.at[pl.ds(base, BPW)], cid_v, sem),
        pltpu.async_copy(sf_hbm.at[pl.ds(base * 3, BPW * 3)], sf_v, sem),
        pltpu.async_copy(nv0_hbm, nv0_v, sem),
        pltpu.async_copy(nv1_hbm, nv1_v, sem),
        pltpu.async_copy(conn_hbm, conn_v, sem),
    ]
    for cp in cps:
        cp.wait()

    iota3 = lax.iota(jnp.int32, L) * 3

    def body(i, _):
        cid = cid_v[pl.ds(i * L, L)]
        c3 = cid * 3
        r3 = i * (3 * L) + iota3
        acc0 = jnp.zeros((L,), jnp.float32)
        acc1 = jnp.zeros((L,), jnp.float32)
        for k in range(3):
            n = plsc.load_gather(conn_v, [c3 + k]) - 1
            w = plsc.load_gather(sf_v, [r3 + k])
            acc0 += w * plsc.load_gather(nv0_v, [n])
            acc1 += w * plsc.load_gather(nv1_v, [n])
        out0_v[pl.ds(i * L, L)] = acc0
        out1_v[pl.ds(i * L, L)] = acc1
        return _

    lax.fori_loop(0, BPW // L, body, None, unroll=1)

    o0 = pltpu.async_copy(out0_v, out_hbm.at[0, pl.ds(base, BPW)], sem)
    o1 = pltpu.async_copy(out1_v, out_hbm.at[1, pl.ds(base, BPW)], sem)
    o0.wait()
    o1.wait()


@functools.partial(
    pl.kernel,
    mesh=plsc.VectorSubcoreMesh(core_axis_name="c", subcore_axis_name="s", num_cores=1),
    out_type=jax.ShapeDtypeStruct((2, B), jnp.float32),
    compiler_params=pltpu.CompilerParams(needs_layout_passes=False),
    scratch_types=[
        pltpu.VMEM((BPW,), jnp.int32),
        pltpu.VMEM((BPW * 3,), jnp.float32),
        pltpu.VMEM((N_NODES,), jnp.float32),
        pltpu.VMEM((N_NODES,), jnp.float32),
        pltpu.VMEM((N_CELLS * 3,), jnp.int32),
        pltpu.VMEM((BPW,), jnp.float32),
        pltpu.VMEM((BPW,), jnp.float32),
        pltpu.SemaphoreType.DMA,
    ],
)
def _interp_kernel(*refs):
    _sc_interp(*refs)


def kernel(x, cell_id, nodal_values, shape_functions, connectivity):
    del x
    sf_flat = shape_functions.reshape(-1)
    nv0 = nodal_values[0, :, 0]
    nv1 = nodal_values[1, :, 0]
    conn_flat = connectivity.reshape(-1)
    return _interp_kernel(cell_id, sf_flat, nv0, nv1, conn_flat)
